# Initial kernel scaffold; baseline (speedup 1.0000x reference)
import jax
import jax.numpy as jnp
from jax.experimental import pallas as pl


def kernel(edge_index, node_ft, edge_sh, edge_feats, batch_idx, num_graphs, Wr1_0, br1_0, Wr2_0, wsh_0, Wlin_0, blin_0, Wr1_1, br1_1, Wr2_1, wsh_1, Wlin_1, blin_1, W_ro1, b_ro1, W_ro2, W0, b0, W2, W4):
    raise NotImplementedError("write your pallas kernel here")


# speedup vs baseline: 1.9553x; 1.9553x over previous
import functools

import jax
import jax.numpy as jnp
import numpy as np
from jax import lax
from jax.experimental import pallas as pl
from jax.experimental.pallas import tpu as pltpu
from jax.experimental.pallas import tpu_sc as plsc

N = 10000
E = 320000
D = 128
SH = 9
EF = 16
MLP = 64
RO = 42
G = 100
AGG_NORM = 32.0

CH = 128
NTILES = 32
NCH_PER_TILE = -(-E // (CH * NTILES))
NCHUNKS = NCH_PER_TILE * NTILES
EP = NCHUNKS * CH
EBLK = 1024
NBLK = 400
NPAD = 10240


def _silu(x):
    return x / (1.0 + jnp.exp(-x))


def _edge_body(ef_ref, sh_ref, w10, b10, w20, g0, w11, b11, w21, g1,
               rg0_ref, rg1_ref):
    i = pl.program_id(0)
    ef = ef_ref[...]
    sh = sh_ref[...]
    rows = lax.broadcasted_iota(jnp.int32, (EBLK, 1), 0) + i * EBLK
    mask = rows < E

    def one(w1, b1, w2, g):
        hid = _silu(jnp.dot(ef, w1[...], preferred_element_type=jnp.float32)
                    + b1[...])
        radial = jnp.dot(hid, w2[...], preferred_element_type=jnp.float32)
        gate = jnp.sum(sh * g[...], axis=1, keepdims=True)
        return jnp.where(mask, radial * gate, 0.0)

    rg0_ref[...] = one(w10, b10, w20, g0)
    rg1_ref[...] = one(w11, b11, w21, g1)


def _edge_precompute(edge_feats, edge_sh, Wr1_0, br1_0, Wr2_0, wsh_0,
                     Wr1_1, br1_1, Wr2_1, wsh_1):
    edge_feats = jnp.zeros((EP, EF), jnp.float32).at[:E].set(edge_feats)
    edge_sh = jnp.zeros((EP, SH), jnp.float32).at[:E].set(edge_sh)
    grid = EP // EBLK
    full = lambda shape: pl.BlockSpec(shape, lambda i: (0,) * len(shape))
    return pl.pallas_call(
        _edge_body,
        grid=(grid,),
        in_specs=[
            pl.BlockSpec((EBLK, EF), lambda i: (i, 0)),
            pl.BlockSpec((EBLK, SH), lambda i: (i, 0)),
            full((EF, MLP)), full((1, MLP)), full((MLP, D)), full((1, SH)),
            full((EF, MLP)), full((1, MLP)), full((MLP, D)), full((1, SH)),
        ],
        out_specs=[
            pl.BlockSpec((EBLK, D), lambda i: (i, 0)),
            pl.BlockSpec((EBLK, D), lambda i: (i, 0)),
        ],
        out_shape=[
            jax.ShapeDtypeStruct((EP, D), jnp.float32),
            jax.ShapeDtypeStruct((EP, D), jnp.float32),
        ],
    )(edge_feats, edge_sh,
      Wr1_0, br1_0.reshape(1, MLP), Wr2_0, wsh_0.reshape(1, SH),
      Wr1_1, br1_1.reshape(1, MLP), Wr2_1, wsh_1.reshape(1, SH))


def _sc_body(h_hbm, rg_hbm, src_hbm, dst_hbm, out_hbm,
             srcv, dstv, rgv, rowsv, acc, sem):
    c = lax.axis_index("c")
    s = lax.axis_index("s")
    wid = c * 16 + s

    def zrow(r, _):
        for k in range(D // 16):
            rowsv[r, pl.ds(k * 16, 16)] = jnp.zeros((16,), jnp.float32)
        return 0

    lax.fori_loop(0, CH, zrow, 0)
    for i in range(NPAD // 16 // CH):
        pltpu.sync_copy(rowsv, acc.at[pl.ds(s * (NPAD // 16) + i * CH, CH)])
    plsc.subcore_barrier()

    def chunk_body(j, _):
        chunk = wid * NCH_PER_TILE + j
        pltpu.sync_copy(src_hbm.at[chunk], srcv)
        pltpu.sync_copy(dst_hbm.at[chunk], dstv)
        pltpu.sync_copy(rg_hbm.at[pl.ds(chunk * CH, CH)], rgv)
        pltpu.async_copy(h_hbm.at[srcv], rowsv, sem).wait()

        def mulrow(r, _):
            for k in range(D // 16):
                sl = pl.ds(k * 16, 16)
                rowsv[r, sl] = rowsv[r, sl] * rgv[r, sl]
            return 0

        lax.fori_loop(0, CH, mulrow, 0)
        pltpu.sync_copy(rowsv, acc.at[dstv], add=True)
        return 0

    lax.fori_loop(0, NCH_PER_TILE, chunk_body, 0)
    plsc.subcore_barrier()

    base = s * 624
    pltpu.sync_copy(acc.at[pl.ds(base, 624)],
                    out_hbm.at[c, pl.ds(base, 624)])

    @pl.when(s == 15)
    def _():
        pltpu.sync_copy(acc.at[pl.ds(9984, 16)],
                        out_hbm.at[c, pl.ds(9984, 16)])


@functools.lru_cache(maxsize=1)
def _sc_kernel():
    return pl.kernel(
        _sc_body,
        out_type=jax.ShapeDtypeStruct((2, N, D), jnp.float32),
        mesh=plsc.VectorSubcoreMesh(core_axis_name="c", subcore_axis_name="s"),
        scratch_types=[
            pltpu.VMEM((CH,), jnp.int32),
            pltpu.VMEM((CH,), jnp.int32),
            pltpu.VMEM((CH, D), jnp.float32),
            pltpu.VMEM((CH, D), jnp.float32),
            pltpu.VMEM_SHARED((NPAD, D), jnp.float32),
            pltpu.SemaphoreType.DMA,
        ],
    )


def _sc_scatter(h, rg, src, dst):
    return _sc_kernel()(h, rg, src, dst)


def _node_body(agg_ref, w_ref, b_ref, res_ref, out_ref):
    a = (agg_ref[0] + agg_ref[1]) * (1.0 / AGG_NORM)
    h = _silu(jnp.dot(a, w_ref[...], preferred_element_type=jnp.float32)
              + b_ref[...])
    out_ref[...] = h + res_ref[...]


def _node_update(agg2, Wlin, blin, res):
    grid = N // NBLK
    full = lambda shape: pl.BlockSpec(shape, lambda i: (0,) * len(shape))
    return pl.pallas_call(
        _node_body,
        grid=(grid,),
        in_specs=[
            pl.BlockSpec((2, NBLK, D), lambda i: (0, i, 0)),
            full((D, D)), full((1, D)),
            pl.BlockSpec((NBLK, D), lambda i: (i, 0)),
        ],
        out_specs=pl.BlockSpec((NBLK, D), lambda i: (i, 0)),
        out_shape=jax.ShapeDtypeStruct((N, D), jnp.float32),
    )(agg2, Wlin, blin.reshape(1, D), res)


def _head_body(agg_ref, h1_ref, w_ref, b_ref, wro1_ref, bro1_ref, wro2_ref,
               bidx_ref, m_ref, c36_ref, ng0_ref, out_ref, sums, cnt):
    i = pl.program_id(0)
    a = (agg_ref[0] + agg_ref[1]) * (1.0 / AGG_NORM)
    h2 = h1_ref[...] + _silu(
        jnp.dot(a, w_ref[...], preferred_element_type=jnp.float32) + b_ref[...])
    o = jnp.dot(
        _silu(jnp.dot(h2, wro1_ref[...], preferred_element_type=jnp.float32)
              + bro1_ref[...]),
        wro2_ref[...], preferred_element_type=jnp.float32)
    onehot = (bidx_ref[...] ==
              lax.broadcasted_iota(jnp.int32, (NBLK, G), 1)).astype(jnp.float32)
    dn = (((0,), (0,)), ((), ()))
    s_part = lax.dot_general(onehot, o, dn,
                             preferred_element_type=jnp.float32)
    c_part = lax.dot_general(onehot, jnp.ones((NBLK, 1), jnp.float32), dn,
                             preferred_element_type=jnp.float32)

    @pl.when(i == 0)
    def _():
        sums[...] = jnp.zeros_like(sums)
        cnt[...] = jnp.zeros_like(cnt)

    sums[...] += s_part
    cnt[...] += c_part

    @pl.when(i == pl.num_programs(0) - 1)
    def _():
        gf = (sums[...] + ng0_ref[...]) / jnp.maximum(cnt[...], 1.0)
        cf = jnp.dot(gf, m_ref[...], preferred_element_type=jnp.float32) \
            + c36_ref[...]
        cols = []
        for i6 in range(6):
            ci = cf[:, 6 * i6:6 * i6 + 6]
            for j6 in range(6):
                cj = cf[:, 6 * j6:6 * j6 + 6]
                col = jnp.sum(ci * cj, axis=1, keepdims=True)
                if i6 == j6:
                    col = col + 1e-3
                cols.append(col)
        out_ref[...] = jnp.concatenate(cols, axis=1)


def _head(agg2, h1, Wlin, blin, W_ro1, b_ro1, W_ro2, bidx2d, M, c36, ng0):
    grid = N // NBLK
    full = lambda shape: pl.BlockSpec(shape, lambda i: (0,) * len(shape))
    return pl.pallas_call(
        _head_body,
        grid=(grid,),
        in_specs=[
            pl.BlockSpec((2, NBLK, D), lambda i: (0, i, 0)),
            pl.BlockSpec((NBLK, D), lambda i: (i, 0)),
            full((D, D)), full((1, D)),
            full((D, D)), full((1, D)), full((D, RO)),
            pl.BlockSpec((NBLK, 1), lambda i: (i, 0)),
            full((RO, 36)), full((1, 36)), full((1, 1)),
        ],
        out_specs=pl.BlockSpec((G, 36), lambda i: (0, 0)),
        out_shape=jax.ShapeDtypeStruct((G, 36), jnp.float32),
        scratch_shapes=[
            pltpu.VMEM((G, RO), jnp.float32),
            pltpu.VMEM((G, 1), jnp.float32),
        ],
    )(agg2, h1, Wlin, blin.reshape(1, D), W_ro1, b_ro1.reshape(1, D), W_ro2,
      bidx2d, M, c36, ng0)


def _build_head_mats(W0, b0, W2, W4):
    A = jnp.zeros((RO, 21), jnp.float32)
    A = A.at[0:4, 0:2].set(W0)
    for i in range(4):
        for o in range(2):
            for m in range(5):
                A = A.at[4 + 5 * i + m, 2 + 5 * o + m].set(W2[i, o])
    for i in range(2):
        for m in range(9):
            A = A.at[24 + 9 * i + m, 12 + m].set(W4[i, 0])
    b21 = jnp.zeros((21,), jnp.float32).at[0:2].set(b0)

    iu0, iu1 = np.triu_indices(6)
    S = np.zeros((21, 36), np.float32)
    for k in range(21):
        ii, jj = int(iu0[k]), int(iu1[k])
        S[k, 6 * ii + jj] = 1.0
        if ii != jj:
            S[k, 6 * jj + ii] = 1.0
    S = jnp.asarray(S)
    return A @ S, (b21 @ S).reshape(1, 36)


def kernel(edge_index, node_ft, edge_sh, edge_feats, batch_idx, num_graphs,
           Wr1_0, br1_0, Wr2_0, wsh_0, Wlin_0, blin_0,
           Wr1_1, br1_1, Wr2_1, wsh_1, Wlin_1, blin_1,
           W_ro1, b_ro1, W_ro2, W0, b0, W2, W4):
    src = jnp.zeros((EP,), jnp.int32).at[:E].set(edge_index[0]).reshape(
        NCHUNKS, CH)
    dst = jnp.zeros((EP,), jnp.int32).at[:E].set(edge_index[1]).reshape(
        NCHUNKS, CH)

    rg0, rg1 = _edge_precompute(edge_feats, edge_sh, Wr1_0, br1_0, Wr2_0,
                                wsh_0, Wr1_1, br1_1, Wr2_1, wsh_1)

    agg0 = _sc_scatter(node_ft, rg0, src, dst)
    h1 = _node_update(agg0, Wlin_0, blin_0, jnp.zeros((N, D), jnp.float32))
    agg1 = _sc_scatter(h1, rg1, src, dst)

    M, c36 = _build_head_mats(W0, b0, W2, W4)
    ng0 = (jnp.asarray(num_graphs, jnp.int32) - G).astype(jnp.float32).reshape(
        1, 1)
    cpos36 = _head(agg1, h1, Wlin_1, blin_1, W_ro1, b_ro1, W_ro2,
                   batch_idx.reshape(N, 1), M, c36, ng0)
    return cpos36.reshape(G, 6, 6)


# speedup vs baseline: 2.7556x; 1.4093x over previous
import functools

import jax
import jax.numpy as jnp
import numpy as np
from jax import lax
from jax.experimental import pallas as pl
from jax.experimental.pallas import tpu as pltpu
from jax.experimental.pallas import tpu_sc as plsc

N = 10000
E = 320000
D = 128
SH = 9
EF = 16
MLP = 64
RO = 42
G = 100
AGG_NORM = 32.0

CH = 64
NTILES = 32
NCH_PER_TILE = 158
NCHUNKS = NCH_PER_TILE * NTILES
EP = NCHUNKS * CH
EBLK = 1024
NBLK = 400
NPAD = 10240


def _silu(x):
    return x / (1.0 + jnp.exp(-x))


def _edge_body(ef_ref, sh_ref, w10, b10, w20, g0, w11, b11, w21, g1,
               rg0_ref, rg1_ref):
    i = pl.program_id(0)
    ef = ef_ref[...]
    sh = sh_ref[...]
    rows = lax.broadcasted_iota(jnp.int32, (EBLK, 1), 0) + i * EBLK
    mask = rows < E

    def one(w1, b1, w2, g):
        hid = _silu(jnp.dot(ef, w1[...], preferred_element_type=jnp.float32)
                    + b1[...])
        radial = jnp.dot(hid, w2[...], preferred_element_type=jnp.float32)
        gate = jnp.sum(sh * g[...], axis=1, keepdims=True)
        return jnp.where(mask, radial * gate, 0.0)

    rg0_ref[...] = one(w10, b10, w20, g0)
    rg1_ref[...] = one(w11, b11, w21, g1)


def _edge_precompute(edge_feats, edge_sh, Wr1_0, br1_0, Wr2_0, wsh_0,
                     Wr1_1, br1_1, Wr2_1, wsh_1):
    grid = EP // EBLK
    last = (E - 1) // EBLK
    clamp = lambda i: (jnp.minimum(i, last), 0)
    full = lambda shape: pl.BlockSpec(shape, lambda i: (0,) * len(shape))
    return pl.pallas_call(
        _edge_body,
        grid=(grid,),
        in_specs=[
            pl.BlockSpec((EBLK, EF), clamp),
            pl.BlockSpec((EBLK, SH), clamp),
            full((EF, MLP)), full((1, MLP)), full((MLP, D)), full((1, SH)),
            full((EF, MLP)), full((1, MLP)), full((MLP, D)), full((1, SH)),
        ],
        out_specs=[
            pl.BlockSpec((EBLK, D), lambda i: (i, 0)),
            pl.BlockSpec((EBLK, D), lambda i: (i, 0)),
        ],
        out_shape=[
            jax.ShapeDtypeStruct((EP, D), jnp.float32),
            jax.ShapeDtypeStruct((EP, D), jnp.float32),
        ],
    )(edge_feats, edge_sh,
      Wr1_0, br1_0.reshape(1, MLP), Wr2_0, wsh_0.reshape(1, SH),
      Wr1_1, br1_1.reshape(1, MLP), Wr2_1, wsh_1.reshape(1, SH))


def _sc_body(h_hbm, rg_hbm, src_hbm, dst_hbm, out_hbm,
             srcv0, srcv1, dstv0, dstv1, rgv0, rgv1, rowsv0, rowsv1, acc,
             ssrc0, ssrc1, sdst0, sdst1, srg0, srg1, sg0, sg1, ssc0, ssc1):
    c = lax.axis_index("c")
    s = lax.axis_index("s")
    wid = c * 16 + s
    base = wid * NCH_PER_TILE
    srcv = [srcv0, srcv1]
    dstv = [dstv0, dstv1]
    rgv = [rgv0, rgv1]
    rowsv = [rowsv0, rowsv1]
    ssrc = [ssrc0, ssrc1]
    sdst = [sdst0, sdst1]
    srg = [srg0, srg1]
    sg = [sg0, sg1]
    ssc = [ssc0, ssc1]

    def zrow(r, _):
        for k in range(D // 16):
            rowsv0[r, pl.ds(k * 16, 16)] = jnp.zeros((16,), jnp.float32)
        return 0

    lax.fori_loop(0, CH, zrow, 0)
    for i in range(NPAD // 16 // CH):
        pltpu.sync_copy(rowsv0, acc.at[pl.ds(s * (NPAD // 16) + i * CH, CH)])
    plsc.subcore_barrier()

    def start_idx(b, chunk):
        pltpu.async_copy(src_hbm.at[chunk], srcv[b], ssrc[b])
        pltpu.async_copy(dst_hbm.at[chunk], dstv[b], sdst[b])
        pltpu.async_copy(rg_hbm.at[pl.ds(chunk * CH, CH)], rgv[b], srg[b])

    def wait_idx_part(b, which):
        if which == 0:
            pltpu.make_async_copy(src_hbm.at[0], srcv[b], ssrc[b]).wait()
        elif which == 1:
            pltpu.make_async_copy(dst_hbm.at[0], dstv[b], sdst[b]).wait()
        else:
            pltpu.make_async_copy(rg_hbm.at[pl.ds(0, CH)], rgv[b],
                                  srg[b]).wait()

    def start_gather(b):
        pltpu.async_copy(h_hbm.at[srcv[b]], rowsv[b], sg[b])

    def wait_gather(b):
        pltpu.make_async_copy(h_hbm.at[srcv[b]], rowsv[b], sg[b]).wait()

    def start_scatter(b):
        pltpu.async_copy(rowsv[b], acc.at[dstv[b]], ssc[b], add=True)

    def wait_scatter(b):
        pltpu.make_async_copy(rowsv[b], acc.at[dstv[b]], ssc[b]).wait()

    start_idx(0, base)
    start_idx(1, base + 1)
    wait_idx_part(0, 0)
    start_gather(0)

    def ring_body(j0, _):
        for b in range(2):
            j = j0 * 2 + b
            bo = 1 - b

            @pl.when(j + 1 < NCH_PER_TILE)
            def _():
                wait_idx_part(bo, 0)

                @pl.when(j >= 1)
                def _():
                    wait_scatter(bo)

                start_gather(bo)

            wait_gather(b)
            wait_idx_part(b, 2)
            wait_idx_part(b, 1)

            def mulrow(r, _):
                for k in range(D // 16):
                    sl = pl.ds(k * 16, 16)
                    rowsv[b][r, sl] = rowsv[b][r, sl] * rgv[b][r, sl]
                return 0

            lax.fori_loop(0, CH, mulrow, 0)
            start_scatter(b)

            @pl.when(j + 2 < NCH_PER_TILE)
            def _():
                start_idx(b, base + j + 2)
        return 0

    lax.fori_loop(0, NCH_PER_TILE // 2, ring_body, 0)
    wait_scatter(0)
    wait_scatter(1)
    plsc.subcore_barrier()

    base = s * 624
    pltpu.sync_copy(acc.at[pl.ds(base, 624)],
                    out_hbm.at[c, pl.ds(base, 624)])

    @pl.when(s == 15)
    def _():
        pltpu.sync_copy(acc.at[pl.ds(9984, 16)],
                        out_hbm.at[c, pl.ds(9984, 16)])


@functools.lru_cache(maxsize=1)
def _sc_kernel():
    return pl.kernel(
        _sc_body,
        out_type=jax.ShapeDtypeStruct((2, N, D), jnp.float32),
        mesh=plsc.VectorSubcoreMesh(core_axis_name="c", subcore_axis_name="s"),
        scratch_types=(
            [pltpu.VMEM((CH,), jnp.int32)] * 4
            + [pltpu.VMEM((CH, D), jnp.float32)] * 4
            + [pltpu.VMEM_SHARED((NPAD, D), jnp.float32)]
            + [pltpu.SemaphoreType.DMA] * 10
        ),
    )


def _sc_scatter(h, rg, src, dst):
    return _sc_kernel()(h, rg, src, dst)


def _node_body(agg_ref, w_ref, b_ref, res_ref, out_ref):
    a = (agg_ref[0] + agg_ref[1]) * (1.0 / AGG_NORM)
    h = _silu(jnp.dot(a, w_ref[...], preferred_element_type=jnp.float32)
              + b_ref[...])
    out_ref[...] = h + res_ref[...]


def _node_update(agg2, Wlin, blin, res):
    grid = N // NBLK
    full = lambda shape: pl.BlockSpec(shape, lambda i: (0,) * len(shape))
    return pl.pallas_call(
        _node_body,
        grid=(grid,),
        in_specs=[
            pl.BlockSpec((2, NBLK, D), lambda i: (0, i, 0)),
            full((D, D)), full((1, D)),
            pl.BlockSpec((NBLK, D), lambda i: (i, 0)),
        ],
        out_specs=pl.BlockSpec((NBLK, D), lambda i: (i, 0)),
        out_shape=jax.ShapeDtypeStruct((N, D), jnp.float32),
    )(agg2, Wlin, blin.reshape(1, D), res)


def _head_body(agg_ref, h1_ref, w_ref, b_ref, wro1_ref, bro1_ref, wro2_ref,
               bidx_ref, m_ref, c36_ref, ng0_ref, out_ref, sums, cnt):
    i = pl.program_id(0)
    a = (agg_ref[0] + agg_ref[1]) * (1.0 / AGG_NORM)
    h2 = h1_ref[...] + _silu(
        jnp.dot(a, w_ref[...], preferred_element_type=jnp.float32) + b_ref[...])
    o = jnp.dot(
        _silu(jnp.dot(h2, wro1_ref[...], preferred_element_type=jnp.float32)
              + bro1_ref[...]),
        wro2_ref[...], preferred_element_type=jnp.float32)
    onehot = (bidx_ref[...] ==
              lax.broadcasted_iota(jnp.int32, (NBLK, G), 1)).astype(jnp.float32)
    dn = (((0,), (0,)), ((), ()))
    s_part = lax.dot_general(onehot, o, dn,
                             preferred_element_type=jnp.float32)
    c_part = lax.dot_general(onehot, jnp.ones((NBLK, 1), jnp.float32), dn,
                             preferred_element_type=jnp.float32)

    @pl.when(i == 0)
    def _():
        sums[...] = jnp.zeros_like(sums)
        cnt[...] = jnp.zeros_like(cnt)

    sums[...] += s_part
    cnt[...] += c_part

    @pl.when(i == pl.num_programs(0) - 1)
    def _():
        gf = (sums[...] + ng0_ref[...]) / jnp.maximum(cnt[...], 1.0)
        cf = jnp.dot(gf, m_ref[...], preferred_element_type=jnp.float32) \
            + c36_ref[...]
        cols = []
        for i6 in range(6):
            ci = cf[:, 6 * i6:6 * i6 + 6]
            for j6 in range(6):
                cj = cf[:, 6 * j6:6 * j6 + 6]
                col = jnp.sum(ci * cj, axis=1, keepdims=True)
                if i6 == j6:
                    col = col + 1e-3
                cols.append(col)
        out_ref[...] = jnp.concatenate(cols, axis=1)


def _head(agg2, h1, Wlin, blin, W_ro1, b_ro1, W_ro2, bidx2d, M, c36, ng0):
    grid = N // NBLK
    full = lambda shape: pl.BlockSpec(shape, lambda i: (0,) * len(shape))
    return pl.pallas_call(
        _head_body,
        grid=(grid,),
        in_specs=[
            pl.BlockSpec((2, NBLK, D), lambda i: (0, i, 0)),
            pl.BlockSpec((NBLK, D), lambda i: (i, 0)),
            full((D, D)), full((1, D)),
            full((D, D)), full((1, D)), full((D, RO)),
            pl.BlockSpec((NBLK, 1), lambda i: (i, 0)),
            full((RO, 36)), full((1, 36)), full((1, 1)),
        ],
        out_specs=pl.BlockSpec((G, 36), lambda i: (0, 0)),
        out_shape=jax.ShapeDtypeStruct((G, 36), jnp.float32),
        scratch_shapes=[
            pltpu.VMEM((G, RO), jnp.float32),
            pltpu.VMEM((G, 1), jnp.float32),
        ],
    )(agg2, h1, Wlin, blin.reshape(1, D), W_ro1, b_ro1.reshape(1, D), W_ro2,
      bidx2d, M, c36, ng0)


def _build_head_mats(W0, b0, W2, W4):
    A = jnp.zeros((RO, 21), jnp.float32)
    A = A.at[0:4, 0:2].set(W0)
    for i in range(4):
        for o in range(2):
            for m in range(5):
                A = A.at[4 + 5 * i + m, 2 + 5 * o + m].set(W2[i, o])
    for i in range(2):
        for m in range(9):
            A = A.at[24 + 9 * i + m, 12 + m].set(W4[i, 0])
    b21 = jnp.zeros((21,), jnp.float32).at[0:2].set(b0)

    iu0, iu1 = np.triu_indices(6)
    S = np.zeros((21, 36), np.float32)
    for k in range(21):
        ii, jj = int(iu0[k]), int(iu1[k])
        S[k, 6 * ii + jj] = 1.0
        if ii != jj:
            S[k, 6 * jj + ii] = 1.0
    S = jnp.asarray(S)
    return A @ S, (b21 @ S).reshape(1, 36)


def kernel(edge_index, node_ft, edge_sh, edge_feats, batch_idx, num_graphs,
           Wr1_0, br1_0, Wr2_0, wsh_0, Wlin_0, blin_0,
           Wr1_1, br1_1, Wr2_1, wsh_1, Wlin_1, blin_1,
           W_ro1, b_ro1, W_ro2, W0, b0, W2, W4):
    src = jnp.zeros((EP,), jnp.int32).at[:E].set(edge_index[0]).reshape(
        NCHUNKS, CH)
    dst = jnp.zeros((EP,), jnp.int32).at[:E].set(edge_index[1]).reshape(
        NCHUNKS, CH)

    rg0, rg1 = _edge_precompute(edge_feats, edge_sh, Wr1_0, br1_0, Wr2_0,
                                wsh_0, Wr1_1, br1_1, Wr2_1, wsh_1)

    agg0 = _sc_scatter(node_ft, rg0, src, dst)
    h1 = _node_update(agg0, Wlin_0, blin_0, jnp.zeros((N, D), jnp.float32))
    agg1 = _sc_scatter(h1, rg1, src, dst)

    M, c36 = _build_head_mats(W0, b0, W2, W4)
    ng0 = (jnp.asarray(num_graphs, jnp.int32) - G).astype(jnp.float32).reshape(
        1, 1)
    cpos36 = _head(agg1, h1, Wlin_1, blin_1, W_ro1, b_ro1, W_ro2,
                   batch_idx.reshape(N, 1), M, c36, ng0)
    return cpos36.reshape(G, 6, 6)


# speedup vs baseline: 3.3226x; 1.2058x over previous
import functools

import jax
import jax.numpy as jnp
import numpy as np
from jax import lax
from jax.experimental import pallas as pl
from jax.experimental.pallas import tpu as pltpu
from jax.experimental.pallas import tpu_sc as plsc

N = 10000
E = 320000
D = 128
SH = 9
EF = 16
MLP = 64
RO = 42
G = 100
AGG_NORM = 32.0

CH = 64
NTILES = 32
NCHUNKS = E // CH
EBLK = 1024
NBLK = 400
NPAD = 10240


def _silu(x):
    return x / (1.0 + jnp.exp(-x))


def _edge_body(ef_ref, sh_ref, w1, b1, w2, g, rg_ref):
    i = pl.program_id(0)
    ef = ef_ref[...]
    sh = sh_ref[...]
    rows = lax.broadcasted_iota(jnp.int32, (EBLK, 1), 0) + i * EBLK
    mask = rows < E
    hid = _silu(jnp.dot(ef, w1[...], preferred_element_type=jnp.float32)
                + b1[...])
    radial = jnp.dot(hid, w2[...], preferred_element_type=jnp.float32)
    gate = jnp.sum(sh * g[...], axis=1, keepdims=True)
    rg_ref[...] = jnp.where(mask, radial * gate, 0.0)


def _edge_precompute(edge_feats, edge_sh, Wr1, br1, Wr2, wsh):
    grid = -(-E // EBLK)
    full = lambda shape: pl.BlockSpec(shape, lambda i: (0,) * len(shape))
    return pl.pallas_call(
        _edge_body,
        grid=(grid,),
        in_specs=[
            pl.BlockSpec((EBLK, EF), lambda i: (i, 0)),
            pl.BlockSpec((EBLK, SH), lambda i: (i, 0)),
            full((EF, MLP)), full((1, MLP)), full((MLP, D)), full((1, SH)),
        ],
        out_specs=pl.BlockSpec((EBLK, D), lambda i: (i, 0)),
        out_shape=jax.ShapeDtypeStruct((E, D), jnp.float32),
    )(edge_feats, edge_sh,
      Wr1, br1.reshape(1, MLP), Wr2, wsh.reshape(1, SH))


def _sc_body(h_hbm, rg_hbm, src_hbm, dst_hbm, out_hbm,
             srcv0, srcv1, dstv0, dstv1, rgv0, rgv1, rowsv0, rowsv1, acc,
             ssrc0, ssrc1, sdst0, sdst1, srg0, srg1, sg0, sg1, ssc0, ssc1):
    c = lax.axis_index("c")
    s = lax.axis_index("s")
    wid = c * 16 + s
    nch = 156 + (wid < 8).astype(jnp.int32)
    base = wid * 156 + jnp.minimum(wid, 8)
    srcv = [srcv0, srcv1]
    dstv = [dstv0, dstv1]
    rgv = [rgv0, rgv1]
    rowsv = [rowsv0, rowsv1]
    ssrc = [ssrc0, ssrc1]
    sdst = [sdst0, sdst1]
    srg = [srg0, srg1]
    sg = [sg0, sg1]
    ssc = [ssc0, ssc1]

    def zrow(r, _):
        for k in range(D // 16):
            rowsv0[r, pl.ds(k * 16, 16)] = jnp.zeros((16,), jnp.float32)
        return 0

    lax.fori_loop(0, CH, zrow, 0)
    for i in range(NPAD // 16 // CH):
        pltpu.sync_copy(rowsv0, acc.at[pl.ds(s * (NPAD // 16) + i * CH, CH)])
    plsc.subcore_barrier()

    def start_idx(b, chunk):
        pltpu.async_copy(src_hbm.at[chunk], srcv[b], ssrc[b])
        pltpu.async_copy(dst_hbm.at[chunk], dstv[b], sdst[b])
        pltpu.async_copy(rg_hbm.at[pl.ds(chunk * CH, CH)], rgv[b], srg[b])

    def wait_idx_part(b, which):
        if which == 0:
            pltpu.make_async_copy(src_hbm.at[0], srcv[b], ssrc[b]).wait()
        elif which == 1:
            pltpu.make_async_copy(dst_hbm.at[0], dstv[b], sdst[b]).wait()
        else:
            pltpu.make_async_copy(rg_hbm.at[pl.ds(0, CH)], rgv[b],
                                  srg[b]).wait()

    def start_gather(b):
        pltpu.async_copy(h_hbm.at[srcv[b]], rowsv[b], sg[b])

    def wait_gather(b):
        pltpu.make_async_copy(h_hbm.at[srcv[b]], rowsv[b], sg[b]).wait()

    def start_scatter(b):
        pltpu.async_copy(rowsv[b], acc.at[dstv[b]], ssc[b], add=True)

    def wait_scatter(b):
        pltpu.make_async_copy(rowsv[b], acc.at[dstv[b]], ssc[b]).wait()

    start_idx(0, base)
    start_idx(1, base + 1)
    wait_idx_part(0, 0)
    start_gather(0)

    def ring_body(j0, _):
        for b in range(2):
            j = j0 * 2 + b
            bo = 1 - b

            @pl.when(j < nch)
            def _():
                @pl.when(j + 1 < nch)
                def _():
                    wait_idx_part(bo, 0)

                    @pl.when(j >= 1)
                    def _():
                        wait_scatter(bo)

                    start_gather(bo)

                wait_gather(b)
                wait_idx_part(b, 2)
                wait_idx_part(b, 1)

                def mulrow(r, _):
                    for k in range(D // 16):
                        sl = pl.ds(k * 16, 16)
                        rowsv[b][r, sl] = rowsv[b][r, sl] * rgv[b][r, sl]
                    return 0

                lax.fori_loop(0, CH, mulrow, 0)
                start_scatter(b)

                @pl.when(j + 2 < nch)
                def _():
                    start_idx(b, base + j + 2)
        return 0

    lax.fori_loop(0, (157 + 1) // 2, ring_body, 0)
    wait_scatter(0)
    wait_scatter(1)
    plsc.subcore_barrier()

    base = s * 624
    pltpu.sync_copy(acc.at[pl.ds(base, 624)],
                    out_hbm.at[c, pl.ds(base, 624)])

    @pl.when(s == 15)
    def _():
        pltpu.sync_copy(acc.at[pl.ds(9984, 16)],
                        out_hbm.at[c, pl.ds(9984, 16)])


@functools.lru_cache(maxsize=1)
def _sc_kernel():
    return pl.kernel(
        _sc_body,
        out_type=jax.ShapeDtypeStruct((2, N, D), jnp.float32),
        mesh=plsc.VectorSubcoreMesh(core_axis_name="c", subcore_axis_name="s"),
        scratch_types=(
            [pltpu.VMEM((CH,), jnp.int32)] * 4
            + [pltpu.VMEM((CH, D), jnp.float32)] * 4
            + [pltpu.VMEM_SHARED((NPAD, D), jnp.float32)]
            + [pltpu.SemaphoreType.DMA] * 10
        ),
    )


def _sc_scatter(h, rg, src, dst):
    return _sc_kernel()(h, rg, src, dst)


def _node_body(agg_ref, w_ref, b_ref, res_ref, out_ref):
    a = (agg_ref[0] + agg_ref[1]) * (1.0 / AGG_NORM)
    h = _silu(jnp.dot(a, w_ref[...], preferred_element_type=jnp.float32)
              + b_ref[...])
    out_ref[...] = h + res_ref[...]


def _node_update(agg2, Wlin, blin, res):
    grid = N // NBLK
    full = lambda shape: pl.BlockSpec(shape, lambda i: (0,) * len(shape))
    return pl.pallas_call(
        _node_body,
        grid=(grid,),
        in_specs=[
            pl.BlockSpec((2, NBLK, D), lambda i: (0, i, 0)),
            full((D, D)), full((1, D)),
            pl.BlockSpec((NBLK, D), lambda i: (i, 0)),
        ],
        out_specs=pl.BlockSpec((NBLK, D), lambda i: (i, 0)),
        out_shape=jax.ShapeDtypeStruct((N, D), jnp.float32),
    )(agg2, Wlin, blin.reshape(1, D), res)


def _head_body(agg_ref, h1_ref, w_ref, b_ref, wro1_ref, bro1_ref, wro2_ref,
               bidx_ref, m_ref, c36_ref, ng0_ref, out_ref, sums, cnt):
    i = pl.program_id(0)
    a = (agg_ref[0] + agg_ref[1]) * (1.0 / AGG_NORM)
    h2 = h1_ref[...] + _silu(
        jnp.dot(a, w_ref[...], preferred_element_type=jnp.float32) + b_ref[...])
    o = jnp.dot(
        _silu(jnp.dot(h2, wro1_ref[...], preferred_element_type=jnp.float32)
              + bro1_ref[...]),
        wro2_ref[...], preferred_element_type=jnp.float32)
    onehot = (bidx_ref[...] ==
              lax.broadcasted_iota(jnp.int32, (NBLK, G), 1)).astype(jnp.float32)
    dn = (((0,), (0,)), ((), ()))
    s_part = lax.dot_general(onehot, o, dn,
                             preferred_element_type=jnp.float32)
    c_part = lax.dot_general(onehot, jnp.ones((NBLK, 1), jnp.float32), dn,
                             preferred_element_type=jnp.float32)

    @pl.when(i == 0)
    def _():
        sums[...] = jnp.zeros_like(sums)
        cnt[...] = jnp.zeros_like(cnt)

    sums[...] += s_part
    cnt[...] += c_part

    @pl.when(i == pl.num_programs(0) - 1)
    def _():
        gf = (sums[...] + ng0_ref[...]) / jnp.maximum(cnt[...], 1.0)
        cf = jnp.dot(gf, m_ref[...], preferred_element_type=jnp.float32) \
            + c36_ref[...]
        cols = []
        for i6 in range(6):
            ci = cf[:, 6 * i6:6 * i6 + 6]
            for j6 in range(6):
                cj = cf[:, 6 * j6:6 * j6 + 6]
                col = jnp.sum(ci * cj, axis=1, keepdims=True)
                if i6 == j6:
                    col = col + 1e-3
                cols.append(col)
        out_ref[...] = jnp.concatenate(cols, axis=1)


def _head(agg2, h1, Wlin, blin, W_ro1, b_ro1, W_ro2, bidx2d, M, c36, ng0):
    grid = N // NBLK
    full = lambda shape: pl.BlockSpec(shape, lambda i: (0,) * len(shape))
    return pl.pallas_call(
        _head_body,
        grid=(grid,),
        in_specs=[
            pl.BlockSpec((2, NBLK, D), lambda i: (0, i, 0)),
            pl.BlockSpec((NBLK, D), lambda i: (i, 0)),
            full((D, D)), full((1, D)),
            full((D, D)), full((1, D)), full((D, RO)),
            pl.BlockSpec((NBLK, 1), lambda i: (i, 0)),
            full((RO, 36)), full((1, 36)), full((1, 1)),
        ],
        out_specs=pl.BlockSpec((G, 36), lambda i: (0, 0)),
        out_shape=jax.ShapeDtypeStruct((G, 36), jnp.float32),
        scratch_shapes=[
            pltpu.VMEM((G, RO), jnp.float32),
            pltpu.VMEM((G, 1), jnp.float32),
        ],
    )(agg2, h1, Wlin, blin.reshape(1, D), W_ro1, b_ro1.reshape(1, D), W_ro2,
      bidx2d, M, c36, ng0)


def _build_head_mats(W0, b0, W2, W4):
    A = jnp.zeros((RO, 21), jnp.float32)
    A = A.at[0:4, 0:2].set(W0)
    for i in range(4):
        for o in range(2):
            for m in range(5):
                A = A.at[4 + 5 * i + m, 2 + 5 * o + m].set(W2[i, o])
    for i in range(2):
        for m in range(9):
            A = A.at[24 + 9 * i + m, 12 + m].set(W4[i, 0])
    b21 = jnp.zeros((21,), jnp.float32).at[0:2].set(b0)

    iu0, iu1 = np.triu_indices(6)
    S = np.zeros((21, 36), np.float32)
    for k in range(21):
        ii, jj = int(iu0[k]), int(iu1[k])
        S[k, 6 * ii + jj] = 1.0
        if ii != jj:
            S[k, 6 * jj + ii] = 1.0
    S = jnp.asarray(S)
    return A @ S, (b21 @ S).reshape(1, 36)


def kernel(edge_index, node_ft, edge_sh, edge_feats, batch_idx, num_graphs,
           Wr1_0, br1_0, Wr2_0, wsh_0, Wlin_0, blin_0,
           Wr1_1, br1_1, Wr2_1, wsh_1, Wlin_1, blin_1,
           W_ro1, b_ro1, W_ro2, W0, b0, W2, W4):
    src = edge_index[0].reshape(NCHUNKS, CH)
    dst = edge_index[1].reshape(NCHUNKS, CH)

    rg0 = _edge_precompute(edge_feats, edge_sh, Wr1_0, br1_0, Wr2_0, wsh_0)
    agg0 = _sc_scatter(node_ft, rg0, src, dst)
    rg1 = _edge_precompute(edge_feats, edge_sh, Wr1_1, br1_1, Wr2_1, wsh_1)
    h1 = _node_update(agg0, Wlin_0, blin_0, jnp.zeros((N, D), jnp.float32))
    agg1 = _sc_scatter(h1, rg1, src, dst)

    M, c36 = _build_head_mats(W0, b0, W2, W4)
    ng0 = (jnp.asarray(num_graphs, jnp.int32) - G).astype(jnp.float32).reshape(
        1, 1)
    cpos36 = _head(agg1, h1, Wlin_1, blin_1, W_ro1, b_ro1, W_ro2,
                   batch_idx.reshape(N, 1), M, c36, ng0)
    return cpos36.reshape(G, 6, 6)


# speedup vs baseline: 3.8612x; 1.1621x over previous
import functools

import jax
import jax.numpy as jnp
import numpy as np
from jax import lax
from jax.experimental import pallas as pl
from jax.experimental.pallas import tpu as pltpu
from jax.experimental.pallas import tpu_sc as plsc

N = 10000
E = 320000
D = 128
SH = 9
EF = 16
MLP = 64
RO = 42
G = 100
AGG_NORM = 32.0

CH = 64
NTILES = 32
NCHUNKS = E // CH
EBLK = 2048
NBLK = 400
NPAD = 10240


def _silu(x):
    return x / (1.0 + jnp.exp(-x))


def _edge_body(ef_ref, sh_ref, w1, b1, w2, g, rg_ref):
    i = pl.program_id(0)
    ef = ef_ref[...]
    sh = sh_ref[...]
    rows = lax.broadcasted_iota(jnp.int32, (EBLK, 1), 0) + i * EBLK
    mask = rows < E
    hid = _silu(jnp.dot(ef, w1[...], preferred_element_type=jnp.float32)
                + b1[...])
    radial = jnp.dot(hid.astype(jnp.bfloat16), w2[...].astype(jnp.bfloat16),
                     preferred_element_type=jnp.float32)
    gate = jnp.sum(sh * g[...], axis=1, keepdims=True)
    rg_ref[...] = jnp.where(mask, radial * gate, 0.0)


def _edge_precompute(edge_feats, edge_sh, Wr1, br1, Wr2, wsh):
    grid = -(-E // EBLK)
    full = lambda shape: pl.BlockSpec(shape, lambda i: (0,) * len(shape))
    return pl.pallas_call(
        _edge_body,
        grid=(grid,),
        in_specs=[
            pl.BlockSpec((EBLK, EF), lambda i: (i, 0)),
            pl.BlockSpec((EBLK, SH), lambda i: (i, 0)),
            full((EF, MLP)), full((1, MLP)), full((MLP, D)), full((1, SH)),
        ],
        out_specs=pl.BlockSpec((EBLK, D), lambda i: (i, 0)),
        out_shape=jax.ShapeDtypeStruct((E, D), jnp.float32),
    )(edge_feats, edge_sh,
      Wr1, br1.reshape(1, MLP), Wr2, wsh.reshape(1, SH))


def _sc_body(h_hbm, rg_hbm, src_hbm, dst_hbm, out_hbm,
             srcv0, srcv1, dstv0, dstv1, rgv0, rgv1, rowsv0, rowsv1, acc,
             ssrc0, ssrc1, sdst0, sdst1, srg0, srg1, sg0, sg1, ssc0, ssc1):
    c = lax.axis_index("c")
    s = lax.axis_index("s")
    wid = c * 16 + s
    nch = 156 + (wid < 8).astype(jnp.int32)
    base = wid * 156 + jnp.minimum(wid, 8)
    srcv = [srcv0, srcv1]
    dstv = [dstv0, dstv1]
    rgv = [rgv0, rgv1]
    rowsv = [rowsv0, rowsv1]
    ssrc = [ssrc0, ssrc1]
    sdst = [sdst0, sdst1]
    srg = [srg0, srg1]
    sg = [sg0, sg1]
    ssc = [ssc0, ssc1]

    def zrow(r, _):
        for k in range(D // 16):
            rowsv0[r, pl.ds(k * 16, 16)] = jnp.zeros((16,), jnp.float32)
        return 0

    lax.fori_loop(0, CH, zrow, 0)
    for i in range(NPAD // 16 // CH):
        pltpu.sync_copy(rowsv0, acc.at[pl.ds(s * (NPAD // 16) + i * CH, CH)])
    plsc.subcore_barrier()

    def start_idx(b, chunk):
        pltpu.async_copy(src_hbm.at[chunk], srcv[b], ssrc[b])
        pltpu.async_copy(dst_hbm.at[chunk], dstv[b], sdst[b])
        pltpu.async_copy(rg_hbm.at[pl.ds(chunk * CH, CH)], rgv[b], srg[b])

    def wait_idx_part(b, which):
        if which == 0:
            pltpu.make_async_copy(src_hbm.at[0], srcv[b], ssrc[b]).wait()
        elif which == 1:
            pltpu.make_async_copy(dst_hbm.at[0], dstv[b], sdst[b]).wait()
        else:
            pltpu.make_async_copy(rg_hbm.at[pl.ds(0, CH)], rgv[b],
                                  srg[b]).wait()

    def start_gather(b):
        pltpu.async_copy(h_hbm.at[srcv[b]], rowsv[b], sg[b])

    def wait_gather(b):
        pltpu.make_async_copy(h_hbm.at[srcv[b]], rowsv[b], sg[b]).wait()

    def start_scatter(b):
        pltpu.async_copy(rowsv[b], acc.at[dstv[b]], ssc[b], add=True)

    def wait_scatter(b):
        pltpu.make_async_copy(rowsv[b], acc.at[dstv[b]], ssc[b]).wait()

    start_idx(0, base)
    start_idx(1, base + 1)
    wait_idx_part(0, 0)
    start_gather(0)

    def ring_body(j0, _):
        for b in range(2):
            j = j0 * 2 + b
            bo = 1 - b

            @pl.when(j < nch)
            def _():
                @pl.when(j + 1 < nch)
                def _():
                    wait_idx_part(bo, 0)

                    @pl.when(j >= 1)
                    def _():
                        wait_scatter(bo)

                    start_gather(bo)

                wait_gather(b)
                wait_idx_part(b, 2)
                wait_idx_part(b, 1)

                def mulrow(r, _):
                    for k in range(D // 16):
                        sl = pl.ds(k * 16, 16)
                        rowsv[b][r, sl] = rowsv[b][r, sl] * rgv[b][r, sl]
                    return 0

                lax.fori_loop(0, CH, mulrow, 0)
                start_scatter(b)

                @pl.when(j + 2 < nch)
                def _():
                    start_idx(b, base + j + 2)
        return 0

    lax.fori_loop(0, (157 + 1) // 2, ring_body, 0)
    wait_scatter(0)
    wait_scatter(1)
    plsc.subcore_barrier()

    base = s * 624
    pltpu.sync_copy(acc.at[pl.ds(base, 624)],
                    out_hbm.at[c, pl.ds(base, 624)])

    @pl.when(s == 15)
    def _():
        pltpu.sync_copy(acc.at[pl.ds(9984, 16)],
                        out_hbm.at[c, pl.ds(9984, 16)])


@functools.lru_cache(maxsize=1)
def _sc_kernel():
    return pl.kernel(
        _sc_body,
        out_type=jax.ShapeDtypeStruct((2, N, D), jnp.float32),
        mesh=plsc.VectorSubcoreMesh(core_axis_name="c", subcore_axis_name="s"),
        scratch_types=(
            [pltpu.VMEM((CH,), jnp.int32)] * 4
            + [pltpu.VMEM((CH, D), jnp.float32)] * 4
            + [pltpu.VMEM_SHARED((NPAD, D), jnp.float32)]
            + [pltpu.SemaphoreType.DMA] * 10
        ),
    )


def _sc_scatter(h, rg, src, dst):
    return _sc_kernel()(h, rg, src, dst)


def _node_body(agg_ref, w_ref, b_ref, res_ref, out_ref):
    a = (agg_ref[0] + agg_ref[1]) * (1.0 / AGG_NORM)
    h = _silu(jnp.dot(a, w_ref[...], preferred_element_type=jnp.float32)
              + b_ref[...])
    out_ref[...] = h + res_ref[...]


def _node_update(agg2, Wlin, blin, res):
    grid = N // NBLK
    full = lambda shape: pl.BlockSpec(shape, lambda i: (0,) * len(shape))
    return pl.pallas_call(
        _node_body,
        grid=(grid,),
        in_specs=[
            pl.BlockSpec((2, NBLK, D), lambda i: (0, i, 0)),
            full((D, D)), full((1, D)),
            pl.BlockSpec((NBLK, D), lambda i: (i, 0)),
        ],
        out_specs=pl.BlockSpec((NBLK, D), lambda i: (i, 0)),
        out_shape=jax.ShapeDtypeStruct((N, D), jnp.float32),
    )(agg2, Wlin, blin.reshape(1, D), res)


def _head_body(agg_ref, h1_ref, w_ref, b_ref, wro1_ref, bro1_ref, wro2_ref,
               bidx_ref, m_ref, c36_ref, ng0_ref, out_ref, sums, cnt):
    i = pl.program_id(0)
    a = (agg_ref[0] + agg_ref[1]) * (1.0 / AGG_NORM)
    h2 = h1_ref[...] + _silu(
        jnp.dot(a, w_ref[...], preferred_element_type=jnp.float32) + b_ref[...])
    o = jnp.dot(
        _silu(jnp.dot(h2, wro1_ref[...], preferred_element_type=jnp.float32)
              + bro1_ref[...]),
        wro2_ref[...], preferred_element_type=jnp.float32)
    onehot = (bidx_ref[...] ==
              lax.broadcasted_iota(jnp.int32, (NBLK, G), 1)).astype(jnp.float32)
    dn = (((0,), (0,)), ((), ()))
    s_part = lax.dot_general(onehot, o, dn,
                             preferred_element_type=jnp.float32)
    c_part = lax.dot_general(onehot, jnp.ones((NBLK, 1), jnp.float32), dn,
                             preferred_element_type=jnp.float32)

    @pl.when(i == 0)
    def _():
        sums[...] = jnp.zeros_like(sums)
        cnt[...] = jnp.zeros_like(cnt)

    sums[...] += s_part
    cnt[...] += c_part

    @pl.when(i == pl.num_programs(0) - 1)
    def _():
        gf = (sums[...] + ng0_ref[...]) / jnp.maximum(cnt[...], 1.0)
        cf = jnp.dot(gf, m_ref[...], preferred_element_type=jnp.float32) \
            + c36_ref[...]
        cols = []
        for i6 in range(6):
            ci = cf[:, 6 * i6:6 * i6 + 6]
            for j6 in range(6):
                cj = cf[:, 6 * j6:6 * j6 + 6]
                col = jnp.sum(ci * cj, axis=1, keepdims=True)
                if i6 == j6:
                    col = col + 1e-3
                cols.append(col)
        out_ref[...] = jnp.concatenate(cols, axis=1)


def _head(agg2, h1, Wlin, blin, W_ro1, b_ro1, W_ro2, bidx2d, M, c36, ng0):
    grid = N // NBLK
    full = lambda shape: pl.BlockSpec(shape, lambda i: (0,) * len(shape))
    return pl.pallas_call(
        _head_body,
        grid=(grid,),
        in_specs=[
            pl.BlockSpec((2, NBLK, D), lambda i: (0, i, 0)),
            pl.BlockSpec((NBLK, D), lambda i: (i, 0)),
            full((D, D)), full((1, D)),
            full((D, D)), full((1, D)), full((D, RO)),
            pl.BlockSpec((NBLK, 1), lambda i: (i, 0)),
            full((RO, 36)), full((1, 36)), full((1, 1)),
        ],
        out_specs=pl.BlockSpec((G, 36), lambda i: (0, 0)),
        out_shape=jax.ShapeDtypeStruct((G, 36), jnp.float32),
        scratch_shapes=[
            pltpu.VMEM((G, RO), jnp.float32),
            pltpu.VMEM((G, 1), jnp.float32),
        ],
    )(agg2, h1, Wlin, blin.reshape(1, D), W_ro1, b_ro1.reshape(1, D), W_ro2,
      bidx2d, M, c36, ng0)


def _build_head_mats(W0, b0, W2, W4):
    A = jnp.zeros((RO, 21), jnp.float32)
    A = A.at[0:4, 0:2].set(W0)
    for i in range(4):
        for o in range(2):
            for m in range(5):
                A = A.at[4 + 5 * i + m, 2 + 5 * o + m].set(W2[i, o])
    for i in range(2):
        for m in range(9):
            A = A.at[24 + 9 * i + m, 12 + m].set(W4[i, 0])
    b21 = jnp.zeros((21,), jnp.float32).at[0:2].set(b0)

    iu0, iu1 = np.triu_indices(6)
    S = np.zeros((21, 36), np.float32)
    for k in range(21):
        ii, jj = int(iu0[k]), int(iu1[k])
        S[k, 6 * ii + jj] = 1.0
        if ii != jj:
            S[k, 6 * jj + ii] = 1.0
    S = jnp.asarray(S)
    return A @ S, (b21 @ S).reshape(1, 36)


def kernel(edge_index, node_ft, edge_sh, edge_feats, batch_idx, num_graphs,
           Wr1_0, br1_0, Wr2_0, wsh_0, Wlin_0, blin_0,
           Wr1_1, br1_1, Wr2_1, wsh_1, Wlin_1, blin_1,
           W_ro1, b_ro1, W_ro2, W0, b0, W2, W4):
    src = edge_index[0].reshape(NCHUNKS, CH)
    dst = edge_index[1].reshape(NCHUNKS, CH)

    rg0 = _edge_precompute(edge_feats, edge_sh, Wr1_0, br1_0, Wr2_0, wsh_0)
    rg1 = _edge_precompute(edge_feats, edge_sh, Wr1_1, br1_1, Wr2_1, wsh_1)
    agg0 = _sc_scatter(node_ft, rg0, src, dst)
    h1 = _node_update(agg0, Wlin_0, blin_0, jnp.zeros((N, D), jnp.float32))
    agg1 = _sc_scatter(h1, rg1, src, dst)

    M, c36 = _build_head_mats(W0, b0, W2, W4)
    ng0 = (jnp.asarray(num_graphs, jnp.int32) - G).astype(jnp.float32).reshape(
        1, 1)
    cpos36 = _head(agg1, h1, Wlin_1, blin_1, W_ro1, b_ro1, W_ro2,
                   batch_idx.reshape(N, 1), M, c36, ng0)
    return cpos36.reshape(G, 6, 6)


# speedup vs baseline: 3.9613x; 1.0259x over previous
import functools

import jax
import jax.numpy as jnp
import numpy as np
from jax import lax
from jax.experimental import pallas as pl
from jax.experimental.pallas import tpu as pltpu
from jax.experimental.pallas import tpu_sc as plsc

N = 10000
E = 320000
D = 128
SH = 9
EF = 16
MLP = 64
RO = 42
G = 100
AGG_NORM = 32.0

CH = 64
NTILES = 32
NCHUNKS = E // CH
EBLK = 2048
NBLK = 2000
NPAD = 10240


def _silu(x):
    return x / (1.0 + jnp.exp(-x))


def _edge_body(ef_ref, sh_ref, w1, b1, w2, g, rg_ref):
    i = pl.program_id(0)
    ef = ef_ref[...]
    sh = sh_ref[...]
    rows = lax.broadcasted_iota(jnp.int32, (EBLK, 1), 0) + i * EBLK
    mask = rows < E
    hid = _silu(jnp.dot(ef, w1[...], preferred_element_type=jnp.float32)
                + b1[...])
    radial = jnp.dot(hid.astype(jnp.bfloat16), w2[...].astype(jnp.bfloat16),
                     preferred_element_type=jnp.float32)
    gate = jnp.sum(sh * g[...], axis=1, keepdims=True)
    rg_ref[...] = jnp.where(mask, radial * gate, 0.0)


def _edge_precompute(edge_feats, edge_sh, Wr1, br1, Wr2, wsh):
    grid = -(-E // EBLK)
    full = lambda shape: pl.BlockSpec(shape, lambda i: (0,) * len(shape))
    return pl.pallas_call(
        _edge_body,
        grid=(grid,),
        in_specs=[
            pl.BlockSpec((EBLK, EF), lambda i: (i, 0)),
            pl.BlockSpec((EBLK, SH), lambda i: (i, 0)),
            full((EF, MLP)), full((1, MLP)), full((MLP, D)), full((1, SH)),
        ],
        out_specs=pl.BlockSpec((EBLK, D), lambda i: (i, 0)),
        out_shape=jax.ShapeDtypeStruct((E, D), jnp.float32),
    )(edge_feats, edge_sh,
      Wr1, br1.reshape(1, MLP), Wr2, wsh.reshape(1, SH))


def _sc_body(h_hbm, rg_hbm, src_hbm, dst_hbm, out_hbm,
             srcv0, srcv1, dstv0, dstv1, rgv0, rgv1, rowsv0, rowsv1, acc,
             ssrc0, ssrc1, sdst0, sdst1, srg0, srg1, sg0, sg1, ssc0, ssc1):
    c = lax.axis_index("c")
    s = lax.axis_index("s")
    wid = c * 16 + s
    nch = 156 + (wid < 8).astype(jnp.int32)
    base = wid * 156 + jnp.minimum(wid, 8)
    srcv = [srcv0, srcv1]
    dstv = [dstv0, dstv1]
    rgv = [rgv0, rgv1]
    rowsv = [rowsv0, rowsv1]
    ssrc = [ssrc0, ssrc1]
    sdst = [sdst0, sdst1]
    srg = [srg0, srg1]
    sg = [sg0, sg1]
    ssc = [ssc0, ssc1]

    def zrow(r, _):
        for k in range(D // 16):
            rowsv0[r, pl.ds(k * 16, 16)] = jnp.zeros((16,), jnp.float32)
        return 0

    lax.fori_loop(0, CH, zrow, 0)
    for i in range(NPAD // 16 // CH):
        pltpu.sync_copy(rowsv0, acc.at[pl.ds(s * (NPAD // 16) + i * CH, CH)])
    plsc.subcore_barrier()

    def start_idx(b, chunk):
        pltpu.async_copy(src_hbm.at[chunk], srcv[b], ssrc[b])
        pltpu.async_copy(dst_hbm.at[chunk], dstv[b], sdst[b])
        pltpu.async_copy(rg_hbm.at[pl.ds(chunk * CH, CH)], rgv[b], srg[b])

    def wait_idx_part(b, which):
        if which == 0:
            pltpu.make_async_copy(src_hbm.at[0], srcv[b], ssrc[b]).wait()
        elif which == 1:
            pltpu.make_async_copy(dst_hbm.at[0], dstv[b], sdst[b]).wait()
        else:
            pltpu.make_async_copy(rg_hbm.at[pl.ds(0, CH)], rgv[b],
                                  srg[b]).wait()

    def start_gather(b):
        pltpu.async_copy(h_hbm.at[srcv[b]], rowsv[b], sg[b])

    def wait_gather(b):
        pltpu.make_async_copy(h_hbm.at[srcv[b]], rowsv[b], sg[b]).wait()

    def start_scatter(b):
        pltpu.async_copy(rowsv[b], acc.at[dstv[b]], ssc[b], add=True)

    def wait_scatter(b):
        pltpu.make_async_copy(rowsv[b], acc.at[dstv[b]], ssc[b]).wait()

    start_idx(0, base)
    start_idx(1, base + 1)
    wait_idx_part(0, 0)
    start_gather(0)

    def ring_body(j0, _):
        for b in range(2):
            j = j0 * 2 + b
            bo = 1 - b

            @pl.when(j < nch)
            def _():
                @pl.when(j + 1 < nch)
                def _():
                    wait_idx_part(bo, 0)

                    @pl.when(j >= 1)
                    def _():
                        wait_scatter(bo)

                    start_gather(bo)

                wait_gather(b)
                wait_idx_part(b, 2)
                wait_idx_part(b, 1)

                def mulrow(r, _):
                    for k in range(D // 16):
                        sl = pl.ds(k * 16, 16)
                        rowsv[b][r, sl] = rowsv[b][r, sl] * rgv[b][r, sl]
                    return 0

                lax.fori_loop(0, CH, mulrow, 0)
                start_scatter(b)

                @pl.when(j + 2 < nch)
                def _():
                    start_idx(b, base + j + 2)
        return 0

    lax.fori_loop(0, (157 + 1) // 2, ring_body, 0)
    wait_scatter(0)
    wait_scatter(1)
    plsc.subcore_barrier()

    base = s * 624
    pltpu.sync_copy(acc.at[pl.ds(base, 624)],
                    out_hbm.at[c, pl.ds(base, 624)])

    @pl.when(s == 15)
    def _():
        pltpu.sync_copy(acc.at[pl.ds(9984, 16)],
                        out_hbm.at[c, pl.ds(9984, 16)])


@functools.lru_cache(maxsize=1)
def _sc_kernel():
    return pl.kernel(
        _sc_body,
        out_type=jax.ShapeDtypeStruct((2, N, D), jnp.float32),
        mesh=plsc.VectorSubcoreMesh(core_axis_name="c", subcore_axis_name="s"),
        scratch_types=(
            [pltpu.VMEM((CH,), jnp.int32)] * 4
            + [pltpu.VMEM((CH, D), jnp.float32)] * 4
            + [pltpu.VMEM_SHARED((NPAD, D), jnp.float32)]
            + [pltpu.SemaphoreType.DMA] * 10
        ),
    )


def _sc_scatter(h, rg, src, dst):
    return _sc_kernel()(h, rg, src, dst)


def _node_body(agg_ref, w_ref, b_ref, res_ref, out_ref):
    a = (agg_ref[0] + agg_ref[1]) * (1.0 / AGG_NORM)
    h = _silu(jnp.dot(a, w_ref[...], preferred_element_type=jnp.float32)
              + b_ref[...])
    out_ref[...] = h + res_ref[...]


def _node_update(agg2, Wlin, blin, res):
    grid = N // NBLK
    full = lambda shape: pl.BlockSpec(shape, lambda i: (0,) * len(shape))
    return pl.pallas_call(
        _node_body,
        grid=(grid,),
        in_specs=[
            pl.BlockSpec((2, NBLK, D), lambda i: (0, i, 0)),
            full((D, D)), full((1, D)),
            pl.BlockSpec((NBLK, D), lambda i: (i, 0)),
        ],
        out_specs=pl.BlockSpec((NBLK, D), lambda i: (i, 0)),
        out_shape=jax.ShapeDtypeStruct((N, D), jnp.float32),
    )(agg2, Wlin, blin.reshape(1, D), res)


def _head_body(agg_ref, h1_ref, w_ref, b_ref, wro1_ref, bro1_ref, wro2_ref,
               bidx_ref, m_ref, c36_ref, ng0_ref, out_ref, sums, cnt):
    i = pl.program_id(0)
    a = (agg_ref[0] + agg_ref[1]) * (1.0 / AGG_NORM)
    h2 = h1_ref[...] + _silu(
        jnp.dot(a, w_ref[...], preferred_element_type=jnp.float32) + b_ref[...])
    o = jnp.dot(
        _silu(jnp.dot(h2, wro1_ref[...], preferred_element_type=jnp.float32)
              + bro1_ref[...]),
        wro2_ref[...], preferred_element_type=jnp.float32)
    onehot = (bidx_ref[...] ==
              lax.broadcasted_iota(jnp.int32, (NBLK, G), 1)).astype(jnp.float32)
    dn = (((0,), (0,)), ((), ()))
    s_part = lax.dot_general(onehot, o, dn,
                             preferred_element_type=jnp.float32)
    c_part = lax.dot_general(onehot, jnp.ones((NBLK, 1), jnp.float32), dn,
                             preferred_element_type=jnp.float32)

    @pl.when(i == 0)
    def _():
        sums[...] = jnp.zeros_like(sums)
        cnt[...] = jnp.zeros_like(cnt)

    sums[...] += s_part
    cnt[...] += c_part

    @pl.when(i == pl.num_programs(0) - 1)
    def _():
        gf = (sums[...] + ng0_ref[...]) / jnp.maximum(cnt[...], 1.0)
        cf = jnp.dot(gf, m_ref[...], preferred_element_type=jnp.float32) \
            + c36_ref[...]
        cols = []
        for i6 in range(6):
            ci = cf[:, 6 * i6:6 * i6 + 6]
            for j6 in range(6):
                cj = cf[:, 6 * j6:6 * j6 + 6]
                col = jnp.sum(ci * cj, axis=1, keepdims=True)
                if i6 == j6:
                    col = col + 1e-3
                cols.append(col)
        out_ref[...] = jnp.concatenate(cols, axis=1)


def _head(agg2, h1, Wlin, blin, W_ro1, b_ro1, W_ro2, bidx2d, M, c36, ng0):
    grid = N // NBLK
    full = lambda shape: pl.BlockSpec(shape, lambda i: (0,) * len(shape))
    return pl.pallas_call(
        _head_body,
        grid=(grid,),
        in_specs=[
            pl.BlockSpec((2, NBLK, D), lambda i: (0, i, 0)),
            pl.BlockSpec((NBLK, D), lambda i: (i, 0)),
            full((D, D)), full((1, D)),
            full((D, D)), full((1, D)), full((D, RO)),
            pl.BlockSpec((NBLK, 1), lambda i: (i, 0)),
            full((RO, 36)), full((1, 36)), full((1, 1)),
        ],
        out_specs=pl.BlockSpec((G, 36), lambda i: (0, 0)),
        out_shape=jax.ShapeDtypeStruct((G, 36), jnp.float32),
        scratch_shapes=[
            pltpu.VMEM((G, RO), jnp.float32),
            pltpu.VMEM((G, 1), jnp.float32),
        ],
    )(agg2, h1, Wlin, blin.reshape(1, D), W_ro1, b_ro1.reshape(1, D), W_ro2,
      bidx2d, M, c36, ng0)


def _build_head_mats(W0, b0, W2, W4):
    A = jnp.zeros((RO, 21), jnp.float32)
    A = A.at[0:4, 0:2].set(W0)
    for i in range(4):
        for o in range(2):
            for m in range(5):
                A = A.at[4 + 5 * i + m, 2 + 5 * o + m].set(W2[i, o])
    for i in range(2):
        for m in range(9):
            A = A.at[24 + 9 * i + m, 12 + m].set(W4[i, 0])
    b21 = jnp.zeros((21,), jnp.float32).at[0:2].set(b0)

    iu0, iu1 = np.triu_indices(6)
    S = np.zeros((21, 36), np.float32)
    for k in range(21):
        ii, jj = int(iu0[k]), int(iu1[k])
        S[k, 6 * ii + jj] = 1.0
        if ii != jj:
            S[k, 6 * jj + ii] = 1.0
    S = jnp.asarray(S)
    return A @ S, (b21 @ S).reshape(1, 36)


def kernel(edge_index, node_ft, edge_sh, edge_feats, batch_idx, num_graphs,
           Wr1_0, br1_0, Wr2_0, wsh_0, Wlin_0, blin_0,
           Wr1_1, br1_1, Wr2_1, wsh_1, Wlin_1, blin_1,
           W_ro1, b_ro1, W_ro2, W0, b0, W2, W4):
    src = edge_index[0].reshape(NCHUNKS, CH)
    dst = edge_index[1].reshape(NCHUNKS, CH)

    rg0 = _edge_precompute(edge_feats, edge_sh, Wr1_0, br1_0, Wr2_0, wsh_0)
    rg1 = _edge_precompute(edge_feats, edge_sh, Wr1_1, br1_1, Wr2_1, wsh_1)
    agg0 = _sc_scatter(node_ft, rg0, src, dst)
    h1 = _node_update(agg0, Wlin_0, blin_0, jnp.zeros((N, D), jnp.float32))
    agg1 = _sc_scatter(h1, rg1, src, dst)

    M, c36 = _build_head_mats(W0, b0, W2, W4)
    ng0 = (jnp.asarray(num_graphs, jnp.int32) - G).astype(jnp.float32).reshape(
        1, 1)
    cpos36 = _head(agg1, h1, Wlin_1, blin_1, W_ro1, b_ro1, W_ro2,
                   batch_idx.reshape(N, 1), M, c36, ng0)
    return cpos36.reshape(G, 6, 6)
